# baseline (speedup 1.0000x reference)
import functools

import jax
import jax.numpy as jnp
from jax import lax
from jax.experimental import pallas as pl
from jax.experimental.pallas import tpu as pltpu
from jax.experimental.pallas import tpu_sc as plsc

_LANES = 128
_DETILE_W = 32768
_HALF_LOG2 = 14


@functools.lru_cache(maxsize=None)
def _make_detile(D, V):
    n_blocks = (V + _DETILE_W - 1) // _DETILE_W
    half = _DETILE_W // 2
    rows_out = n_blocks * half

    eye = None

    def detile_kernel(tw_ref, out_ref):
        ident = jax.lax.broadcasted_iota(jnp.int32, (D, D), 0) == \
            jax.lax.broadcasted_iota(jnp.int32, (D, D), 1)
        identf = ident.astype(jnp.float32)
        t = jax.lax.dot_general(
            tw_ref[...], identf,
            (((0,), (0,)), ((), ())),
            preferred_element_type=jnp.float32,
        )
        out_ref[:, 0:D] = t[0:half, :]
        out_ref[:, D : 2 * D] = t[half : 2 * half, :]

    return pl.pallas_call(
        detile_kernel,
        grid=(n_blocks,),
        in_specs=[pl.BlockSpec((D, _DETILE_W), lambda i: (0, i))],
        out_specs=pl.BlockSpec((half, 2 * D), lambda i: (i, 0)),
        out_shape=jax.ShapeDtypeStruct((rows_out, 2 * D), jnp.float32),
    )


@functools.lru_cache(maxsize=None)
def _make_gather(VP, D, batch, seq):
    info = plsc.get_sparse_core_info()
    NC, NS = info.num_cores, info.num_subcores
    NW = NC * NS
    assert batch % NW == 0
    rows_per_w = batch // NW
    C = seq
    SEQ_PAD = (seq + 7) // 8 * 8
    CP = SEQ_PAD
    NBUF = 4
    AHEAD = 2
    assert rows_per_w % NBUF == 0 and AHEAD < NBUF
    n_groups = rows_per_w // NBUF
    mesh = plsc.VectorSubcoreMesh(core_axis_name="c", subcore_axis_name="s")

    @functools.partial(
        pl.kernel,
        mesh=mesh,
        compiler_params=pltpu.CompilerParams(use_tc_tiling_on_sc=False),
        out_type=jax.ShapeDtypeStruct((batch, SEQ_PAD, 2 * D), jnp.float32),
        scratch_types=[
            pltpu.VMEM((rows_per_w * CP,), jnp.int32),
            pltpu.VMEM((NBUF, C, D), jnp.float32),
        ]
        + [pltpu.SemaphoreType.DMA] * (2 * NBUF),
    )
    def gather_kernel(table_hbm, idx_hbm, out_hbm, idxk_v, rows_v, *sems):
        gsems, wsems = sems[:NBUF], sems[NBUF:]
        wid = lax.axis_index("s") * NC + lax.axis_index("c")
        base = wid * rows_per_w
        n_idx = rows_per_w * CP
        pltpu.sync_copy(idx_hbm.at[pl.ds(wid * n_idx, n_idx)], idxk_v)

        def remap_body(m):
            r = idxk_v[pl.ds(m * 16, 16)]
            k = (
                (r & jnp.int32(-_DETILE_W))
                | ((r & jnp.int32(_DETILE_W // 2 - 1)) << 1)
                | ((r >> _HALF_LOG2) & jnp.int32(1))
            )
            idxk_v[pl.ds(m * 16, 16)] = k

        pl.loop(0, n_idx // 16)(remap_body)

        def start_gather(g, b):
            return pltpu.async_copy(
                table_hbm.at[idxk_v.at[pl.ds(g * CP, C)]], rows_v.at[b], gsems[b]
            )

        def wait_gather(g, b):
            pltpu.make_async_copy(
                table_hbm.at[idxk_v.at[pl.ds(g * CP, C)]], rows_v.at[b], gsems[b]
            ).wait()

        def start_writeout(g, b):
            return pltpu.async_copy(
                rows_v.at[b],
                out_hbm.at[base + g].at[pl.ds(0, C), pl.ds(0, D)],
                wsems[b],
            )

        def wait_writeout(g, b):
            pltpu.make_async_copy(
                rows_v.at[b],
                out_hbm.at[base + g].at[pl.ds(0, C), pl.ds(0, D)],
                wsems[b],
            ).wait()

        for p in range(AHEAD):
            start_gather(p, p)
        for b in range(NBUF):
            p = b + AHEAD
            if p < rows_per_w:
                if p >= NBUF:
                    wait_writeout(p - NBUF, p % NBUF)
                start_gather(p, p % NBUF)
            wait_gather(b, b)
            start_writeout(b, b)

        def group_body(m):
            g0 = m * NBUF
            for b in range(NBUF):
                g = g0 + b
                p = g + AHEAD
                bp = (b + AHEAD) % NBUF
                wait_writeout(p - NBUF, bp)
                start_gather(p, bp)
                wait_gather(g, b)
                start_writeout(g, b)

        if n_groups > 2:
            pl.loop(1, n_groups - 1)(group_body)

        if n_groups > 1:
            g0 = (n_groups - 1) * NBUF
            for b in range(NBUF):
                g = g0 + b
                p = g + AHEAD
                bp = (b + AHEAD) % NBUF
                if p < rows_per_w:
                    wait_writeout(p - NBUF, bp)
                    start_gather(p, bp)
                wait_gather(g, b)
                start_writeout(g, b)
        for b in range(NBUF):
            g = (n_groups - 1) * NBUF + b
            wait_writeout(g, b)

    return gather_kernel


def kernel(inputs, word_embeddings):
    batch, seq = inputs.shape
    V, D = word_embeddings.shape
    seq_pad = (seq + 7) // 8 * 8
    idx = jnp.pad(inputs.astype(jnp.int32), ((0, 0), (0, seq_pad - seq)))
    wt2 = _make_detile(D, V)(word_embeddings.T)
    wt_lin = wt2.reshape(-1, D)
    out = _make_gather(wt_lin.shape[0], D, batch, seq)(wt_lin, idx.reshape(-1))
    return out[:, :seq, :D]


# speedup vs baseline: 1.0032x; 1.0032x over previous
import functools

import jax
import jax.numpy as jnp
from jax import lax
from jax.experimental import pallas as pl
from jax.experimental.pallas import tpu as pltpu
from jax.experimental.pallas import tpu_sc as plsc

_LANES = 128
_DETILE_W = 32768
_HALF_LOG2 = 14


@functools.lru_cache(maxsize=None)
def _make_detile(D, V):
    n_blocks = (V + _DETILE_W - 1) // _DETILE_W
    half = _DETILE_W // 2
    rows_out = n_blocks * half

    def detile_kernel(tw_ref, out_ref):
        t = tw_ref[...].T
        out_ref[:, 0:D] = t[0:half, :]
        out_ref[:, D : 2 * D] = t[half : 2 * half, :]

    return pl.pallas_call(
        detile_kernel,
        grid=(n_blocks,),
        in_specs=[pl.BlockSpec((D, _DETILE_W), lambda i: (0, i))],
        out_specs=pl.BlockSpec((half, 2 * D), lambda i: (i, 0)),
        out_shape=jax.ShapeDtypeStruct((rows_out, 2 * D), jnp.float32),
    )


@functools.lru_cache(maxsize=None)
def _make_gather(VP, D, batch, seq):
    info = plsc.get_sparse_core_info()
    NC, NS = info.num_cores, info.num_subcores
    NW = NC * NS
    assert batch % NW == 0
    rows_per_w = batch // NW
    C = seq
    SEQ_PAD = (seq + 7) // 8 * 8
    CP = SEQ_PAD
    NBUF = 4
    AHEAD = 2
    assert rows_per_w % NBUF == 0 and AHEAD < NBUF
    n_groups = rows_per_w // NBUF
    mesh = plsc.VectorSubcoreMesh(core_axis_name="c", subcore_axis_name="s")

    @functools.partial(
        pl.kernel,
        mesh=mesh,
        compiler_params=pltpu.CompilerParams(use_tc_tiling_on_sc=False),
        out_type=jax.ShapeDtypeStruct((batch, SEQ_PAD, 2 * D), jnp.float32),
        scratch_types=[
            pltpu.VMEM((rows_per_w * CP,), jnp.int32),
            pltpu.VMEM((NBUF, C, D), jnp.float32),
        ]
        + [pltpu.SemaphoreType.DMA] * (2 * NBUF),
    )
    def gather_kernel(table_hbm, idx_hbm, out_hbm, idxk_v, rows_v, *sems):
        gsems, wsems = sems[:NBUF], sems[NBUF:]
        wid = lax.axis_index("s") * NC + lax.axis_index("c")
        base = wid * rows_per_w
        n_idx = rows_per_w * CP
        pltpu.sync_copy(idx_hbm.at[pl.ds(wid * n_idx, n_idx)], idxk_v)

        def remap_body(m):
            r = idxk_v[pl.ds(m * 16, 16)]
            k = (
                (r & jnp.int32(-_DETILE_W))
                | ((r & jnp.int32(_DETILE_W // 2 - 1)) << 1)
                | ((r >> _HALF_LOG2) & jnp.int32(1))
            )
            idxk_v[pl.ds(m * 16, 16)] = k

        pl.loop(0, n_idx // 16)(remap_body)

        def start_gather(g, b):
            return pltpu.async_copy(
                table_hbm.at[idxk_v.at[pl.ds(g * CP, C)]], rows_v.at[b], gsems[b]
            )

        def wait_gather(g, b):
            pltpu.make_async_copy(
                table_hbm.at[idxk_v.at[pl.ds(g * CP, C)]], rows_v.at[b], gsems[b]
            ).wait()

        def start_writeout(g, b):
            return pltpu.async_copy(
                rows_v.at[b],
                out_hbm.at[base + g].at[pl.ds(0, C), pl.ds(0, D)],
                wsems[b],
            )

        def wait_writeout(g, b):
            pltpu.make_async_copy(
                rows_v.at[b],
                out_hbm.at[base + g].at[pl.ds(0, C), pl.ds(0, D)],
                wsems[b],
            ).wait()

        for p in range(AHEAD):
            start_gather(p, p)
        for b in range(NBUF):
            p = b + AHEAD
            if p < rows_per_w:
                if p >= NBUF:
                    wait_writeout(p - NBUF, p % NBUF)
                start_gather(p, p % NBUF)
            wait_gather(b, b)
            start_writeout(b, b)

        def group_body(m):
            g0 = m * NBUF
            for b in range(NBUF):
                g = g0 + b
                p = g + AHEAD
                bp = (b + AHEAD) % NBUF
                wait_writeout(p - NBUF, bp)
                start_gather(p, bp)
                wait_gather(g, b)
                start_writeout(g, b)

        if n_groups > 2:
            pl.loop(1, n_groups - 1)(group_body)

        if n_groups > 1:
            g0 = (n_groups - 1) * NBUF
            for b in range(NBUF):
                g = g0 + b
                p = g + AHEAD
                bp = (b + AHEAD) % NBUF
                if p < rows_per_w:
                    wait_writeout(p - NBUF, bp)
                    start_gather(p, bp)
                wait_gather(g, b)
                start_writeout(g, b)
        for b in range(NBUF):
            g = (n_groups - 1) * NBUF + b
            wait_writeout(g, b)

    return gather_kernel


def kernel(inputs, word_embeddings):
    batch, seq = inputs.shape
    V, D = word_embeddings.shape
    seq_pad = (seq + 7) // 8 * 8
    idx = jnp.pad(inputs.astype(jnp.int32), ((0, 0), (0, seq_pad - seq)))
    wt2 = _make_detile(D, V)(word_embeddings.T)
    wt_lin = wt2.reshape(-1, D)
    out = _make_gather(wt_lin.shape[0], D, batch, seq)(wt_lin, idx.reshape(-1))
    return out[:, :seq, :D]
